# Initial kernel scaffold; baseline (speedup 1.0000x reference)
import jax
import jax.numpy as jnp
from jax.experimental import pallas as pl


def kernel(x, edge_index, W1, b1, W2, b2, W3, b3):
    raise NotImplementedError("write your pallas kernel here")


# speedup vs baseline: 6.4096x; 6.4096x over previous
import functools

import jax
import jax.numpy as jnp
from jax import lax
from jax.experimental import pallas as pl
from jax.experimental.pallas import tpu as pltpu
from jax.experimental.pallas import tpu_sc as plsc

N = 10000
NR = 10112
RPT = 632
E = 320000
CHUNK = 128
NCHUNK = 2560
EP = NCHUNK * CHUNK
DCH = NCHUNK // 32
RING = 8
CH = 128


def _mesh():
  return plsc.VectorSubcoreMesh(core_axis_name="c", subcore_axis_name="s")


DLANES = 32


def _deg_kernel(dst2):
  @functools.partial(
      pl.kernel,
      out_type=jax.ShapeDtypeStruct((2 * NR, DLANES), jnp.float32),
      mesh=_mesh(),
      scratch_types=[
          pltpu.VMEM((DCH, CHUNK), jnp.int32),
          pltpu.VMEM((CHUNK, DLANES), jnp.float32),
          pltpu.VMEM((CHUNK, DLANES), jnp.float32),
          pltpu.VMEM_SHARED((NR, DLANES), jnp.float32),
          pltpu.SemaphoreType.DMA,
          pltpu.SemaphoreType.DMA,
      ],
  )
  def degk(dst_hbm, out, didx, o0, o1, acc, s0, s1):
    c = lax.axis_index("c")
    s = lax.axis_index("s")
    row0 = s * RPT
    base = (c * 16 + s) * DCH
    pltpu.sync_copy(dst_hbm.at[pl.ds(base, DCH)], didx)

    z16 = jnp.zeros((16,), jnp.float32)
    o16 = jnp.ones((16,), jnp.float32)

    def _fill(buf, val):
      def _row(r, carry):
        for g in range(DLANES // 16):
          buf[r, pl.ds(g * 16, 16)] = val
        return carry

      lax.fori_loop(0, CHUNK, _row, 0)

    _fill(o0, z16)
    for k in range(4):
      pltpu.sync_copy(o0, acc.at[pl.ds(row0 + k * CHUNK, CHUNK)])
    pltpu.sync_copy(o0.at[pl.ds(0, 120)], acc.at[pl.ds(row0 + 512, 120)])
    _fill(o0, o16)
    _fill(o1, o16)
    plsc.subcore_barrier()

    def _sstart(j, buf, sem):
      pltpu.async_copy(buf, acc.at[didx.at[j]], sem, add=True)

    def _swait(j, buf, sem):
      pltpu.make_async_copy(buf, acc.at[didx.at[j]], sem).wait()

    def _scat(u, carry):
      j0 = 2 * u

      @pl.when(u >= 1)
      def _():
        _swait(j0 - 2, o0, s0)

      _sstart(j0, o0, s0)

      @pl.when(u >= 1)
      def _():
        _swait(j0 - 1, o1, s1)

      _sstart(j0 + 1, o1, s1)
      return carry

    lax.fori_loop(0, DCH // 2, _scat, 0)
    _swait(DCH - 2, o0, s0)
    _swait(DCH - 1, o1, s1)
    plsc.subcore_barrier()

    out0 = c * NR + row0
    for k in range(2):
      pltpu.sync_copy(acc.at[pl.ds(row0 + k * CHUNK, CHUNK)], o0)
      pltpu.sync_copy(o0, out.at[pl.ds(out0 + k * CHUNK, CHUNK)])
      pltpu.sync_copy(acc.at[pl.ds(row0 + (k + 2) * CHUNK, CHUNK)], o1)
      pltpu.sync_copy(o1, out.at[pl.ds(out0 + (k + 2) * CHUNK, CHUNK)])
    pltpu.sync_copy(acc.at[pl.ds(row0 + 512, 120)], o0.at[pl.ds(0, 120)])
    pltpu.sync_copy(o0.at[pl.ds(0, 120)], out.at[pl.ds(out0 + 512, 120)])

  return degk(dst2)


STRIPE = 16


def _spmm(hp, src2, dst2):
  tch = NCHUNK // 16
  nstripe = tch // STRIPE

  @functools.partial(
      pl.kernel,
      out_type=jax.ShapeDtypeStruct((2 * NR, CH), jnp.float32),
      mesh=_mesh(),
      scratch_types=[
          pltpu.VMEM((STRIPE, CHUNK), jnp.int32),
          pltpu.VMEM((STRIPE, CHUNK), jnp.int32),
          pltpu.VMEM((CHUNK, CH), jnp.float32),
          pltpu.VMEM((CHUNK, CH), jnp.float32),
          pltpu.VMEM_SHARED((NR, CH), jnp.float32),
          pltpu.SemaphoreType.DMA,
          pltpu.SemaphoreType.DMA,
          pltpu.SemaphoreType.DMA,
          pltpu.SemaphoreType.DMA,
      ],
  )
  def spmm(hp_hbm, src_hbm, dst_hbm, out, sidx, didx, a0, a1,
           acc, ga0, ga1, sa0, sa1):
    c = lax.axis_index("c")
    s = lax.axis_index("s")
    row0 = s * RPT
    base = c * NCHUNK + s * tch

    z16 = jnp.zeros((16,), jnp.float32)

    def _zero_a0(r, carry):
      for g in range(CH // 16):
        a0[r, pl.ds(g * 16, 16)] = z16
      return carry

    def _gstart(j, buf, sem):
      pltpu.async_copy(hp_hbm.at[sidx.at[j]], buf, sem)

    def _gwait(j, buf, sem):
      pltpu.make_async_copy(hp_hbm.at[sidx.at[j]], buf, sem).wait()

    def _sstart(j, buf, sem):
      pltpu.async_copy(buf, acc.at[didx.at[j]], sem, add=True)

    def _swait(j, buf, sem):
      pltpu.make_async_copy(buf, acc.at[didx.at[j]], sem).wait()

    lax.fori_loop(0, CHUNK, _zero_a0, 0)
    for k in range(4):
      pltpu.sync_copy(a0, acc.at[pl.ds(row0 + k * CHUNK, CHUNK)])
    pltpu.sync_copy(a0.at[pl.ds(0, 120)], acc.at[pl.ds(row0 + 512, 120)])
    plsc.subcore_barrier()

    for t in range(nstripe):
      pltpu.sync_copy(src_hbm.at[pl.ds(base + t * STRIPE, STRIPE)], sidx)
      pltpu.sync_copy(dst_hbm.at[pl.ds(base + t * STRIPE, STRIPE)], didx)

      def _iter(u, carry):
        j0 = 2 * u

        @pl.when(u >= 1)
        def _():
          _swait(j0 - 2, a0, sa0)

        _gstart(j0, a0, ga0)

        @pl.when(u >= 1)
        def _():
          _swait(j0 - 1, a1, sa1)

        _gstart(j0 + 1, a1, ga1)
        _gwait(j0, a0, ga0)
        _sstart(j0, a0, sa0)
        _gwait(j0 + 1, a1, ga1)
        _sstart(j0 + 1, a1, sa1)
        return carry

      lax.fori_loop(0, STRIPE // 2, _iter, 0)
      _swait(STRIPE - 2, a0, sa0)
      _swait(STRIPE - 1, a1, sa1)

    plsc.subcore_barrier()

    out0 = c * NR + row0
    for k in range(2):
      pltpu.sync_copy(acc.at[pl.ds(row0 + k * CHUNK, CHUNK)], a0)
      pltpu.sync_copy(a0, out.at[pl.ds(out0 + k * CHUNK, CHUNK)])
      pltpu.sync_copy(acc.at[pl.ds(row0 + (k + 2) * CHUNK, CHUNK)], a1)
      pltpu.sync_copy(a1, out.at[pl.ds(out0 + (k + 2) * CHUNK, CHUNK)])
    pltpu.sync_copy(acc.at[pl.ds(row0 + 512, 120)], a0.at[pl.ds(0, 120)])
    pltpu.sync_copy(a0.at[pl.ds(0, 120)], out.at[pl.ds(out0 + 512, 120)])

  return spmm(hp, src2, dst2)


GB = 16


def _dis_of(d_ref):
  d = d_ref[0, :, 0:1] + d_ref[1, :, 0:1] + 1.0
  return lax.rsqrt(d)


def _split(h, o_ref, nslice):
  for k in range(nslice):
    o_ref[k] = h[:, k * CH:(k + 1) * CH]


def _cat(ref, nslice):
  return jnp.concatenate([ref[k] for k in range(nslice)], axis=1)


def _tc_first(xp, w1, deg2):
  def body(x_ref, w_ref, d_ref, o_ref):
    dis = _dis_of(d_ref)
    h = jnp.dot(x_ref[...], w_ref[...], preferred_element_type=jnp.float32)
    _split(h * dis, o_ref, 2)

  return pl.pallas_call(
      body,
      grid=(GB,),
      in_specs=[
          pl.BlockSpec((RPT, 128), lambda i: (i, 0)),
          pl.BlockSpec((128, 256), lambda i: (0, 0)),
          pl.BlockSpec((2, RPT, DLANES), lambda i: (0, i, 0)),
      ],
      out_specs=pl.BlockSpec((2, RPT, CH), lambda i: (0, i, 0)),
      out_shape=jax.ShapeDtypeStruct((2, NR, CH), jnp.float32),
  )(xp, w1, deg2)


def _tc_mid(acc, hp, deg2, b, w, ns_out):
  def body(a_ref, h_ref, d_ref, b_ref, w_ref, o_ref):
    dis = _dis_of(d_ref)
    a = _cat(a_ref, 2)
    hh = _cat(h_ref, 2)
    xl = jnp.maximum(dis * (a + hh) + b_ref[...], 0.0)
    h2 = jnp.dot(xl, w_ref[...], preferred_element_type=jnp.float32) * dis
    _split(h2, o_ref, ns_out)

  return pl.pallas_call(
      body,
      grid=(GB,),
      in_specs=[
          pl.BlockSpec((2, RPT, CH), lambda i: (0, i, 0)),
          pl.BlockSpec((2, RPT, CH), lambda i: (0, i, 0)),
          pl.BlockSpec((2, RPT, DLANES), lambda i: (0, i, 0)),
          pl.BlockSpec((1, 256), lambda i: (0, 0)),
          pl.BlockSpec((256, ns_out * CH), lambda i: (0, 0)),
      ],
      out_specs=pl.BlockSpec((ns_out, RPT, CH), lambda i: (0, i, 0)),
      out_shape=jax.ShapeDtypeStruct((ns_out, NR, CH), jnp.float32),
  )(acc, hp, deg2, b, w)


def _tc_final(acc, hp, deg2, b3p):
  def body(a_ref, h_ref, d_ref, b_ref, o_ref):
    dis = _dis_of(d_ref)
    o_ref[...] = dis * (a_ref[0] + h_ref[0]) + b_ref[...]

  return pl.pallas_call(
      body,
      grid=(GB,),
      in_specs=[
          pl.BlockSpec((1, RPT, CH), lambda i: (0, i, 0)),
          pl.BlockSpec((1, RPT, CH), lambda i: (0, i, 0)),
          pl.BlockSpec((2, RPT, DLANES), lambda i: (0, i, 0)),
          pl.BlockSpec((1, 128), lambda i: (0, 0)),
      ],
      out_specs=pl.BlockSpec((RPT, 128), lambda i: (i, 0)),
      out_shape=jax.ShapeDtypeStruct((NR, 128), jnp.float32),
  )(acc, hp, deg2, b3p)


def kernel(x, edge_index, W1, b1, W2, b2, W3, b3):
  src = edge_index[0].astype(jnp.int32)
  dst = edge_index[1].astype(jnp.int32)
  pad = jnp.full((EP - E,), N, jnp.int32)
  src2 = jnp.concatenate([src, pad]).reshape(NCHUNK, CHUNK)
  dst2 = jnp.concatenate([dst, pad]).reshape(NCHUNK, CHUNK)
  src_sl = jnp.concatenate([src2, src2 + NR])
  dst_b = jnp.concatenate([dst2, dst2])

  xp = jnp.pad(x, ((0, NR - N), (0, 0)))
  w3p = jnp.pad(W3, ((0, 0), (0, 128 - W3.shape[1])))
  b1r = b1.reshape(1, -1)
  b2r = b2.reshape(1, -1)
  b3p = jnp.pad(b3, (0, 128 - b3.shape[0])).reshape(1, -1)

  deg2 = _deg_kernel(dst2).reshape(2, NR, DLANES)
  h1 = _tc_first(xp, W1, deg2)
  acc1 = _spmm(h1.reshape(2 * NR, CH), src_sl, dst_b).reshape(2, NR, CH)
  h2 = _tc_mid(acc1, h1, deg2, b1r, W2, 2)
  acc2 = _spmm(h2.reshape(2 * NR, CH), src_sl, dst_b).reshape(2, NR, CH)
  h3 = _tc_mid(acc2, h2, deg2, b2r, w3p, 1)
  h3d = jnp.concatenate([h3, h3]).reshape(2 * NR, CH)
  acc3 = _spmm(h3d, src_sl, dst_b).reshape(2, NR, CH)
  out = _tc_final(acc3[0:1], h3, deg2, b3p)
  return out[:N, :121]


# speedup vs baseline: 7.0485x; 1.0997x over previous
import functools

import jax
import jax.numpy as jnp
from jax import lax
from jax.experimental import pallas as pl
from jax.experimental.pallas import tpu as pltpu
from jax.experimental.pallas import tpu_sc as plsc

N = 10000
NR = 10112
RPT = 632
E = 320000
CHUNK = 128
NCHUNK = 2560
EP = NCHUNK * CHUNK
DCH = NCHUNK // 32
RING = 8
CH = 128


def _mesh():
  return plsc.VectorSubcoreMesh(core_axis_name="c", subcore_axis_name="s")


DLANES = 32


def _deg_kernel(dst2):
  @functools.partial(
      pl.kernel,
      out_type=jax.ShapeDtypeStruct((2 * NR, DLANES), jnp.float32),
      mesh=_mesh(),
      scratch_types=[
          pltpu.VMEM((DCH, CHUNK), jnp.int32),
          pltpu.VMEM((CHUNK, DLANES), jnp.float32),
          pltpu.VMEM((CHUNK, DLANES), jnp.float32),
          pltpu.VMEM_SHARED((NR, DLANES), jnp.float32),
          pltpu.SemaphoreType.DMA,
          pltpu.SemaphoreType.DMA,
      ],
  )
  def degk(dst_hbm, out, didx, o0, o1, acc, s0, s1):
    c = lax.axis_index("c")
    s = lax.axis_index("s")
    row0 = s * RPT
    base = (c * 16 + s) * DCH
    pltpu.sync_copy(dst_hbm.at[pl.ds(base, DCH)], didx)

    z16 = jnp.zeros((16,), jnp.float32)
    o16 = jnp.ones((16,), jnp.float32)

    def _fill(buf, val):
      def _row(r, carry):
        for g in range(DLANES // 16):
          buf[r, pl.ds(g * 16, 16)] = val
        return carry

      lax.fori_loop(0, CHUNK, _row, 0)

    _fill(o0, z16)
    for k in range(4):
      pltpu.sync_copy(o0, acc.at[pl.ds(row0 + k * CHUNK, CHUNK)])
    pltpu.sync_copy(o0.at[pl.ds(0, 120)], acc.at[pl.ds(row0 + 512, 120)])
    _fill(o0, o16)
    _fill(o1, o16)
    plsc.subcore_barrier()

    def _sstart(j, buf, sem):
      pltpu.async_copy(buf, acc.at[didx.at[j]], sem, add=True)

    def _swait(j, buf, sem):
      pltpu.make_async_copy(buf, acc.at[didx.at[j]], sem).wait()

    def _scat(u, carry):
      j0 = 2 * u

      @pl.when(u >= 1)
      def _():
        _swait(j0 - 2, o0, s0)

      _sstart(j0, o0, s0)

      @pl.when(u >= 1)
      def _():
        _swait(j0 - 1, o1, s1)

      _sstart(j0 + 1, o1, s1)
      return carry

    lax.fori_loop(0, DCH // 2, _scat, 0)
    _swait(DCH - 2, o0, s0)
    _swait(DCH - 1, o1, s1)
    plsc.subcore_barrier()

    out0 = c * NR + row0
    for k in range(2):
      pltpu.sync_copy(acc.at[pl.ds(row0 + k * CHUNK, CHUNK)], o0)
      pltpu.sync_copy(o0, out.at[pl.ds(out0 + k * CHUNK, CHUNK)])
      pltpu.sync_copy(acc.at[pl.ds(row0 + (k + 2) * CHUNK, CHUNK)], o1)
      pltpu.sync_copy(o1, out.at[pl.ds(out0 + (k + 2) * CHUNK, CHUNK)])
    pltpu.sync_copy(acc.at[pl.ds(row0 + 512, 120)], o0.at[pl.ds(0, 120)])
    pltpu.sync_copy(o0.at[pl.ds(0, 120)], out.at[pl.ds(out0 + 512, 120)])

  return degk(dst2)


STRIPE = 16


def _spmm(hp, src2, dst2, cpc=NCHUNK):
  tch = cpc // 16
  nstripe = tch // STRIPE

  @functools.partial(
      pl.kernel,
      out_type=jax.ShapeDtypeStruct((2 * NR, CH), jnp.float32),
      mesh=_mesh(),
      scratch_types=[
          pltpu.VMEM((STRIPE, CHUNK), jnp.int32),
          pltpu.VMEM((STRIPE, CHUNK), jnp.int32),
          pltpu.VMEM((CHUNK, CH), jnp.float32),
          pltpu.VMEM((CHUNK, CH), jnp.float32),
          pltpu.VMEM_SHARED((NR, CH), jnp.float32),
          pltpu.SemaphoreType.DMA,
          pltpu.SemaphoreType.DMA,
          pltpu.SemaphoreType.DMA,
          pltpu.SemaphoreType.DMA,
      ],
  )
  def spmm(hp_hbm, src_hbm, dst_hbm, out, sidx, didx, a0, a1,
           acc, ga0, ga1, sa0, sa1):
    c = lax.axis_index("c")
    s = lax.axis_index("s")
    row0 = s * RPT
    base = c * cpc + s * tch

    z16 = jnp.zeros((16,), jnp.float32)

    def _zero_a0(r, carry):
      for g in range(CH // 16):
        a0[r, pl.ds(g * 16, 16)] = z16
      return carry

    def _gstart(j, buf, sem):
      pltpu.async_copy(hp_hbm.at[sidx.at[j]], buf, sem)

    def _gwait(j, buf, sem):
      pltpu.make_async_copy(hp_hbm.at[sidx.at[j]], buf, sem).wait()

    def _sstart(j, buf, sem):
      pltpu.async_copy(buf, acc.at[didx.at[j]], sem, add=True)

    def _swait(j, buf, sem):
      pltpu.make_async_copy(buf, acc.at[didx.at[j]], sem).wait()

    lax.fori_loop(0, CHUNK, _zero_a0, 0)
    for k in range(4):
      pltpu.sync_copy(a0, acc.at[pl.ds(row0 + k * CHUNK, CHUNK)])
    pltpu.sync_copy(a0.at[pl.ds(0, 120)], acc.at[pl.ds(row0 + 512, 120)])
    plsc.subcore_barrier()

    for t in range(nstripe):
      pltpu.sync_copy(src_hbm.at[pl.ds(base + t * STRIPE, STRIPE)], sidx)
      pltpu.sync_copy(dst_hbm.at[pl.ds(base + t * STRIPE, STRIPE)], didx)

      def _iter(u, carry):
        j0 = 2 * u

        @pl.when(u >= 1)
        def _():
          _swait(j0 - 2, a0, sa0)

        _gstart(j0, a0, ga0)

        @pl.when(u >= 1)
        def _():
          _swait(j0 - 1, a1, sa1)

        _gstart(j0 + 1, a1, ga1)
        _gwait(j0, a0, ga0)
        _sstart(j0, a0, sa0)
        _gwait(j0 + 1, a1, ga1)
        _sstart(j0 + 1, a1, sa1)
        return carry

      lax.fori_loop(0, STRIPE // 2, _iter, 0)
      _swait(STRIPE - 2, a0, sa0)
      _swait(STRIPE - 1, a1, sa1)

    plsc.subcore_barrier()

    out0 = c * NR + row0
    for k in range(2):
      pltpu.sync_copy(acc.at[pl.ds(row0 + k * CHUNK, CHUNK)], a0)
      pltpu.sync_copy(a0, out.at[pl.ds(out0 + k * CHUNK, CHUNK)])
      pltpu.sync_copy(acc.at[pl.ds(row0 + (k + 2) * CHUNK, CHUNK)], a1)
      pltpu.sync_copy(a1, out.at[pl.ds(out0 + (k + 2) * CHUNK, CHUNK)])
    pltpu.sync_copy(acc.at[pl.ds(row0 + 512, 120)], a0.at[pl.ds(0, 120)])
    pltpu.sync_copy(a0.at[pl.ds(0, 120)], out.at[pl.ds(out0 + 512, 120)])

  return spmm(hp, src2, dst2)


GB = 16


def _dis_of(d_ref):
  d = d_ref[0, :, 0:1] + d_ref[1, :, 0:1] + 1.0
  return lax.rsqrt(d)


def _split(h, o_ref, nslice):
  for k in range(nslice):
    o_ref[k] = h[:, k * CH:(k + 1) * CH]


def _cat(ref, nslice):
  return jnp.concatenate([ref[k] for k in range(nslice)], axis=1)


def _tc_first(xp, w1, deg2):
  def body(x_ref, w_ref, d_ref, o_ref):
    dis = _dis_of(d_ref)
    h = jnp.dot(x_ref[...], w_ref[...], preferred_element_type=jnp.float32)
    _split(h * dis, o_ref, 2)

  return pl.pallas_call(
      body,
      grid=(GB,),
      in_specs=[
          pl.BlockSpec((RPT, 128), lambda i: (i, 0)),
          pl.BlockSpec((128, 256), lambda i: (0, 0)),
          pl.BlockSpec((2, RPT, DLANES), lambda i: (0, i, 0)),
      ],
      out_specs=pl.BlockSpec((2, RPT, CH), lambda i: (0, i, 0)),
      out_shape=jax.ShapeDtypeStruct((2, NR, CH), jnp.float32),
  )(xp, w1, deg2)


def _tc_mid(acc, hp, deg2, b, w, ns_out):
  def body(a_ref, h_ref, d_ref, b_ref, w_ref, o_ref):
    dis = _dis_of(d_ref)
    a = _cat(a_ref, 2)
    hh = _cat(h_ref, 2)
    xl = jnp.maximum(dis * (a + hh) + b_ref[...], 0.0)
    h2 = jnp.dot(xl, w_ref[...], preferred_element_type=jnp.float32) * dis
    _split(h2, o_ref, ns_out)

  return pl.pallas_call(
      body,
      grid=(GB,),
      in_specs=[
          pl.BlockSpec((2, RPT, CH), lambda i: (0, i, 0)),
          pl.BlockSpec((2, RPT, CH), lambda i: (0, i, 0)),
          pl.BlockSpec((2, RPT, DLANES), lambda i: (0, i, 0)),
          pl.BlockSpec((1, 256), lambda i: (0, 0)),
          pl.BlockSpec((256, ns_out * CH), lambda i: (0, 0)),
      ],
      out_specs=pl.BlockSpec((ns_out, RPT, CH), lambda i: (0, i, 0)),
      out_shape=jax.ShapeDtypeStruct((ns_out, NR, CH), jnp.float32),
  )(acc, hp, deg2, b, w)


def _tc_final(acc, hp, deg2, b3p):
  def body(a_ref, h_ref, d_ref, b_ref, o_ref):
    dis = _dis_of(d_ref)
    o_ref[...] = dis * (a_ref[0] + a_ref[1] + h_ref[0]) + b_ref[...]

  return pl.pallas_call(
      body,
      grid=(GB,),
      in_specs=[
          pl.BlockSpec((2, RPT, CH), lambda i: (0, i, 0)),
          pl.BlockSpec((1, RPT, CH), lambda i: (0, i, 0)),
          pl.BlockSpec((2, RPT, DLANES), lambda i: (0, i, 0)),
          pl.BlockSpec((1, 128), lambda i: (0, 0)),
      ],
      out_specs=pl.BlockSpec((RPT, 128), lambda i: (i, 0)),
      out_shape=jax.ShapeDtypeStruct((NR, 128), jnp.float32),
  )(acc, hp, deg2, b3p)


def kernel(x, edge_index, W1, b1, W2, b2, W3, b3):
  src = edge_index[0].astype(jnp.int32)
  dst = edge_index[1].astype(jnp.int32)
  pad = jnp.full((EP - E,), N, jnp.int32)
  src2 = jnp.concatenate([src, pad]).reshape(NCHUNK, CHUNK)
  dst2 = jnp.concatenate([dst, pad]).reshape(NCHUNK, CHUNK)
  src_sl = jnp.concatenate([src2, src2 + NR])
  dst_b = jnp.concatenate([dst2, dst2])

  xp = jnp.pad(x, ((0, NR - N), (0, 0)))
  w3p = jnp.pad(W3, ((0, 0), (0, 128 - W3.shape[1])))
  b1r = b1.reshape(1, -1)
  b2r = b2.reshape(1, -1)
  b3p = jnp.pad(b3, (0, 128 - b3.shape[0])).reshape(1, -1)

  deg2 = _deg_kernel(dst2).reshape(2, NR, DLANES)
  h1 = _tc_first(xp, W1, deg2)
  acc1 = _spmm(h1.reshape(2 * NR, CH), src_sl, dst_b).reshape(2, NR, CH)
  h2 = _tc_mid(acc1, h1, deg2, b1r, W2, 2)
  acc2 = _spmm(h2.reshape(2 * NR, CH), src_sl, dst_b).reshape(2, NR, CH)
  h3 = _tc_mid(acc2, h2, deg2, b2r, w3p, 1)
  acc3 = _spmm(h3.reshape(NR, CH), src2, dst2, NCHUNK // 2).reshape(2, NR, CH)
  out = _tc_final(acc3, h3, deg2, b3p)
  return out[:N, :121]
